# baseline (speedup 1.0000x reference)
import functools

import jax
from jax import lax
import jax.numpy as jnp
from jax.experimental import pallas as pl
from jax.experimental.pallas import tpu as pltpu
from jax.experimental.pallas import tpu_sc as plsc

NC = 2
NS = 16
NW = NC * NS
LANES = 16
BATCH = 128

_MESH = plsc.VectorSubcoreMesh(core_axis_name="c", subcore_axis_name="s")
_SC_PARAMS = pltpu.CompilerParams(use_tc_tiling_on_sc=False)


def _deg_body(ktile, np_, col_hbm, zeros_hbm, ones_hbm, out_hbm,
              col_v, ones_v, acc_sh, sem):
    cid = lax.axis_index("c")
    sid = lax.axis_index("s")
    chunk = np_ // NS
    base = sid * chunk
    wid = cid * NS + sid
    setup = [
        pltpu.async_copy(zeros_hbm.at[pl.ds(base, chunk)],
                         acc_sh.at[pl.ds(base, chunk)], sem),
        pltpu.async_copy(ones_hbm, ones_v, sem),
        pltpu.async_copy(col_hbm.at[wid], col_v, sem),
    ]
    for c in setup:
        c.wait()
    plsc.subcore_barrier()
    adds = [pltpu.async_copy(ones_v, acc_sh.at[col_v.at[j]], sem, add=True)
            for j in range(ktile)]
    for c in adds:
        c.wait()
    plsc.subcore_barrier()
    pltpu.sync_copy(acc_sh.at[pl.ds(base, chunk)],
                    out_hbm.at[cid].at[pl.ds(base, chunk)])


def _agg_body(ktile, np_, row_hbm, col_hbm, y_hbm, zeros_hbm, out_hbm,
              row_v, col_v, rows_v, y_sh, acc_sh, gsem, ssem):
    cid = lax.axis_index("c")
    sid = lax.axis_index("s")
    chunk = np_ // NS
    base = sid * chunk
    wid = cid * NS + sid
    setup = [
        pltpu.async_copy(zeros_hbm.at[pl.ds(base, chunk)],
                         acc_sh.at[pl.ds(base, chunk)], gsem),
        pltpu.async_copy(y_hbm.at[pl.ds(base, chunk)],
                         y_sh.at[pl.ds(base, chunk)], gsem),
        pltpu.async_copy(row_hbm.at[wid], row_v, gsem),
        pltpu.async_copy(col_hbm.at[wid], col_v, gsem),
    ]
    for c in setup:
        c.wait()
    plsc.subcore_barrier()
    gathers = [pltpu.async_copy(y_sh.at[row_v.at[j]], rows_v.at[j], gsem)
               for j in range(ktile)]
    for c in gathers:
        c.wait()
    adds = [pltpu.async_copy(rows_v.at[j], acc_sh.at[col_v.at[j]], ssem,
                             add=True)
            for j in range(ktile)]
    for c in adds:
        c.wait()
    plsc.subcore_barrier()
    pltpu.sync_copy(acc_sh.at[pl.ds(base, chunk)],
                    out_hbm.at[cid].at[pl.ds(base, chunk)])


def _mm_kernel(x_ref, w_ref, d_ref, o_ref):
    xw = jax.lax.dot_general(
        x_ref[...], w_ref[...], (((1,), (0,)), ((), ())),
        preferred_element_type=jnp.float32,
        precision=jax.lax.Precision.DEFAULT)
    dinv = jax.lax.rsqrt(1.0 + d_ref[0] + d_ref[1])
    o_ref[...] = xw * dinv


def _out_kernel(s_ref, d_ref, y_ref, b_ref, o_ref):
    dinv = jax.lax.rsqrt(1.0 + d_ref[0] + d_ref[1])
    t = dinv * (s_ref[0] + s_ref[1] + y_ref[...]) + b_ref[...]
    o_ref[...] = jnp.maximum(t, 0.0)


def kernel(x, edge_index, W, b):
    n, d_in = x.shape
    d_out = W.shape[1]
    e = edge_index.shape[1]

    ktile = pl.cdiv(e, NW * BATCH)
    e_pad = NW * ktile * BATCH
    np_ = ((n + 1 + NS * 8 - 1) // (NS * 8)) * (NS * 8)

    row = jnp.concatenate(
        [edge_index[0], jnp.full((e_pad - e,), n, jnp.int32)]
    ).reshape(NW, ktile, BATCH)
    col = jnp.concatenate(
        [edge_index[1], jnp.full((e_pad - e,), n, jnp.int32)]
    ).reshape(NW, ktile, BATCH)
    w_pad = jnp.pad(W, ((0, 0), (0, LANES - d_out)))
    b_row = jnp.tile(jnp.pad(b, (0, LANES - d_out)), 128 // LANES).reshape(1, 128)
    zeros_np = jnp.zeros((np_, LANES), jnp.float32)
    ones_blk = jnp.ones((BATCH, LANES), jnp.float32)

    deg_parts = pl.kernel(
        functools.partial(_deg_body, ktile, np_),
        out_type=jax.ShapeDtypeStruct((NC, np_, LANES), jnp.float32),
        mesh=_MESH,
        compiler_params=_SC_PARAMS,
        scratch_types=[
            pltpu.VMEM((ktile, BATCH), jnp.int32),
            pltpu.VMEM((BATCH, LANES), jnp.float32),
            pltpu.VMEM_SHARED((np_, LANES), jnp.float32),
            pltpu.SemaphoreType.DMA,
        ],
    )(col, zeros_np, ones_blk)

    rblk = 2000
    y = pl.pallas_call(
        _mm_kernel,
        grid=(n // rblk,),
        in_specs=[
            pl.BlockSpec((rblk, d_in), lambda i: (i, 0)),
            pl.BlockSpec((d_in, LANES), lambda i: (0, 0)),
            pl.BlockSpec((NC, rblk, LANES), lambda i: (0, i, 0)),
        ],
        out_specs=pl.BlockSpec((rblk, LANES), lambda i: (i, 0)),
        out_shape=jax.ShapeDtypeStruct((np_, LANES), jnp.float32),
    )(x, w_pad, deg_parts)

    nv = np_ * LANES // 128
    deg_v = deg_parts.reshape(NC, nv, 128)
    y_v = y.reshape(nv, 128)

    s_parts = pl.kernel(
        functools.partial(_agg_body, ktile, np_),
        out_type=jax.ShapeDtypeStruct((NC, np_, LANES), jnp.float32),
        mesh=_MESH,
        compiler_params=_SC_PARAMS,
        scratch_types=[
            pltpu.VMEM((ktile, BATCH), jnp.int32),
            pltpu.VMEM((ktile, BATCH), jnp.int32),
            pltpu.VMEM((ktile, BATCH, LANES), jnp.float32),
            pltpu.VMEM_SHARED((np_, LANES), jnp.float32),
            pltpu.VMEM_SHARED((np_, LANES), jnp.float32),
            pltpu.SemaphoreType.DMA,
            pltpu.SemaphoreType.DMA,
        ],
    )(row, col, y, zeros_np)

    out_v = pl.pallas_call(
        _out_kernel,
        out_shape=jax.ShapeDtypeStruct((nv, 128), jnp.float32),
    )(s_parts.reshape(NC, nv, 128), deg_v, y_v, b_row)

    return out_v.reshape(np_, LANES)[:n, :d_out]


# speedup vs baseline: 1.2555x; 1.2555x over previous
import functools

import jax
from jax import lax
import jax.numpy as jnp
from jax.experimental import pallas as pl
from jax.experimental.pallas import tpu as pltpu
from jax.experimental.pallas import tpu_sc as plsc

NC = 2
NS = 16
NW = NC * NS
LANES = 8
BATCH = 128

_MESH = plsc.VectorSubcoreMesh(core_axis_name="c", subcore_axis_name="s")
_SC_PARAMS = pltpu.CompilerParams(use_tc_tiling_on_sc=False)


def _deg_body(ktile, np_, col_hbm, zeros_hbm, ones_hbm, out_hbm,
              col_v, ones_v, acc_sh, sem):
    cid = lax.axis_index("c")
    sid = lax.axis_index("s")
    chunk = np_ // NS
    base = sid * chunk
    wid = cid * NS + sid
    setup = [
        pltpu.async_copy(zeros_hbm.at[pl.ds(base, chunk)],
                         acc_sh.at[pl.ds(base, chunk)], sem),
        pltpu.async_copy(ones_hbm, ones_v, sem),
        pltpu.async_copy(col_hbm.at[wid], col_v, sem),
    ]
    for c in setup:
        c.wait()
    plsc.subcore_barrier()
    adds = [pltpu.async_copy(ones_v, acc_sh.at[col_v.at[j]], sem, add=True)
            for j in range(ktile)]
    for c in adds:
        c.wait()
    plsc.subcore_barrier()
    pltpu.sync_copy(acc_sh.at[pl.ds(base, chunk)],
                    out_hbm.at[cid].at[pl.ds(base, chunk)])


def _agg_body(ktile, np_, row_hbm, col_hbm, y_hbm, zeros_hbm, out_hbm,
              row_v, col_v, rows_v, y_sh, acc_sh, gsem, ssem):
    cid = lax.axis_index("c")
    sid = lax.axis_index("s")
    chunk = np_ // NS
    base = sid * chunk
    wid = cid * NS + sid
    setup = [
        pltpu.async_copy(zeros_hbm.at[pl.ds(base, chunk)],
                         acc_sh.at[pl.ds(base, chunk)], gsem),
        pltpu.async_copy(y_hbm.at[pl.ds(base, chunk)],
                         y_sh.at[pl.ds(base, chunk)], gsem),
        pltpu.async_copy(row_hbm.at[wid], row_v, gsem),
        pltpu.async_copy(col_hbm.at[wid], col_v, gsem),
    ]
    for c in setup:
        c.wait()
    plsc.subcore_barrier()
    gathers = [pltpu.async_copy(y_sh.at[row_v.at[j]], rows_v.at[j], gsem)
               for j in range(ktile)]
    for c in gathers:
        c.wait()
    adds = [pltpu.async_copy(rows_v.at[j], acc_sh.at[col_v.at[j]], ssem,
                             add=True)
            for j in range(ktile)]
    for c in adds:
        c.wait()
    plsc.subcore_barrier()
    pltpu.sync_copy(acc_sh.at[pl.ds(base, chunk)],
                    out_hbm.at[cid].at[pl.ds(base, chunk)])


def _mm_kernel(x_ref, w_ref, o_ref):
    o_ref[...] = jax.lax.dot_general(
        x_ref[...], w_ref[...], (((1,), (0,)), ((), ())),
        preferred_element_type=jnp.float32,
        precision=jax.lax.Precision.DEFAULT)


def _y_kernel(xw_ref, d_ref, o_ref):
    deg = 1.0 + d_ref[0] + d_ref[1]
    o_ref[...] = xw_ref[...] * jax.lax.rsqrt(deg)


def _out_kernel(s_ref, d_ref, y_ref, b_ref, o_ref):
    dinv = jax.lax.rsqrt(1.0 + d_ref[0] + d_ref[1])
    t = dinv * (s_ref[0] + s_ref[1] + y_ref[...]) + b_ref[...]
    o_ref[...] = jnp.maximum(t, 0.0)


def kernel(x, edge_index, W, b):
    n, d_in = x.shape
    d_out = W.shape[1]
    e = edge_index.shape[1]

    ktile = pl.cdiv(e, NW * BATCH)
    e_pad = NW * ktile * BATCH
    np_ = ((n + 1 + NS * 8 - 1) // (NS * 8)) * (NS * 8)

    row = jnp.concatenate(
        [edge_index[0], jnp.full((e_pad - e,), n, jnp.int32)]
    ).reshape(NW, ktile, BATCH)
    col = jnp.concatenate(
        [edge_index[1], jnp.full((e_pad - e,), n, jnp.int32)]
    ).reshape(NW, ktile, BATCH)
    w_pad = jnp.pad(W, ((0, 0), (0, LANES - d_out)))
    b_row = jnp.tile(jnp.pad(b, (0, LANES - d_out)), 128 // LANES).reshape(1, 128)
    zeros_np = jnp.zeros((np_, LANES), jnp.float32)
    ones_blk = jnp.ones((BATCH, LANES), jnp.float32)

    deg_parts = pl.kernel(
        functools.partial(_deg_body, ktile, np_),
        out_type=jax.ShapeDtypeStruct((NC, np_, LANES), jnp.float32),
        mesh=_MESH,
        compiler_params=_SC_PARAMS,
        scratch_types=[
            pltpu.VMEM((ktile, BATCH), jnp.int32),
            pltpu.VMEM((BATCH, LANES), jnp.float32),
            pltpu.VMEM_SHARED((np_, LANES), jnp.float32),
            pltpu.SemaphoreType.DMA,
        ],
    )(col, zeros_np, ones_blk)

    rblk = 2000
    xw = pl.pallas_call(
        _mm_kernel,
        grid=(n // rblk,),
        in_specs=[
            pl.BlockSpec((rblk, d_in), lambda i: (i, 0)),
            pl.BlockSpec((d_in, LANES), lambda i: (0, 0)),
        ],
        out_specs=pl.BlockSpec((rblk, LANES), lambda i: (i, 0)),
        out_shape=jax.ShapeDtypeStruct((np_, LANES), jnp.float32),
    )(x, w_pad)

    nv = np_ * LANES // 128
    deg_v = deg_parts.reshape(NC, nv, 128)
    y_v = pl.pallas_call(
        _y_kernel,
        out_shape=jax.ShapeDtypeStruct((nv, 128), jnp.float32),
    )(xw.reshape(nv, 128), deg_v)
    y = y_v.reshape(np_, LANES)

    s_parts = pl.kernel(
        functools.partial(_agg_body, ktile, np_),
        out_type=jax.ShapeDtypeStruct((NC, np_, LANES), jnp.float32),
        mesh=_MESH,
        compiler_params=_SC_PARAMS,
        scratch_types=[
            pltpu.VMEM((ktile, BATCH), jnp.int32),
            pltpu.VMEM((ktile, BATCH), jnp.int32),
            pltpu.VMEM((ktile, BATCH, LANES), jnp.float32),
            pltpu.VMEM_SHARED((np_, LANES), jnp.float32),
            pltpu.VMEM_SHARED((np_, LANES), jnp.float32),
            pltpu.SemaphoreType.DMA,
            pltpu.SemaphoreType.DMA,
        ],
    )(row, col, y, zeros_np)

    out_v = pl.pallas_call(
        _out_kernel,
        out_shape=jax.ShapeDtypeStruct((nv, 128), jnp.float32),
    )(s_parts.reshape(NC, nv, 128), deg_v, y_v, b_row)

    return out_v.reshape(np_, LANES)[:n, :d_out]
